# baseline (speedup 1.0000x reference)
import functools

import jax
import jax.numpy as jnp
from jax import lax
from jax.experimental import pallas as pl
from jax.experimental.pallas import tpu as pltpu
from jax.experimental.pallas import tpu_sc as plsc

_NC = 2
_NS = 16


def _sc_gather_body(b_per_w, tok_hbm, mu_hbm, ls_hbm, phi_hbm,
                    mu_out, ls_out, phi_out,
                    idx_v, mu_v, ls_v, phi_v, sem):
    wid = lax.axis_index("s") * _NC + lax.axis_index("c")
    base = wid * b_per_w
    pltpu.sync_copy(tok_hbm.at[pl.ds(base, b_per_w)], idx_v)
    d_mu = pltpu.async_copy(mu_hbm.at[idx_v], mu_v, sem)
    d_ls = pltpu.async_copy(ls_hbm.at[idx_v], ls_v, sem)
    d_phi = pltpu.async_copy(phi_hbm.at[idx_v], phi_v, sem)
    d_mu.wait()
    d_ls.wait()
    d_phi.wait()
    pltpu.sync_copy(mu_v, mu_out.at[pl.ds(base, b_per_w)])
    pltpu.sync_copy(ls_v, ls_out.at[pl.ds(base, b_per_w)])
    pltpu.sync_copy(phi_v, phi_out.at[pl.ds(base, b_per_w)])


def _sc_gather(tok_flat, mu_table, log_sigma_diag, phi_pad):
    n = tok_flat.shape[0]
    k = mu_table.shape[1]
    p = phi_pad.shape[1]
    nw = _NC * _NS
    b_per_w = n // nw
    mesh = plsc.VectorSubcoreMesh(core_axis_name="c", subcore_axis_name="s",
                                  num_cores=_NC, num_subcores=_NS)
    kern = pl.kernel(
        functools.partial(_sc_gather_body, b_per_w),
        out_type=(
            jax.ShapeDtypeStruct((n, k), jnp.float32),
            jax.ShapeDtypeStruct((n, k), jnp.float32),
            jax.ShapeDtypeStruct((n, p), jnp.float32),
        ),
        mesh=mesh,
        scratch_types=[
            pltpu.VMEM((b_per_w,), jnp.int32),
            pltpu.VMEM((b_per_w, k), jnp.float32),
            pltpu.VMEM((b_per_w, k), jnp.float32),
            pltpu.VMEM((b_per_w, p), jnp.float32),
            pltpu.SemaphoreType.DMA,
        ],
        compiler_params=pltpu.CompilerParams(use_tc_tiling_on_sc=False),
    )
    return kern(tok_flat, mu_table, log_sigma_diag, phi_pad)


def _expand_body(ls_ref, out_ref):
    k = ls_ref.shape[1]
    sig = jnp.exp(ls_ref[...])
    j = lax.broadcasted_iota(jnp.int32, (k, k * k), 0)
    c = lax.broadcasted_iota(jnp.int32, (k, k * k), 1)
    e = jnp.where(c == (k + 1) * j, 1.0, 0.0).astype(jnp.float32)
    out_ref[...] = lax.dot_general(sig, e, (((1,), (0,)), ((), ())),
                                   preferred_element_type=jnp.float32)


def _expand_diag(ls_flat, block):
    n, k = ls_flat.shape
    grid = n // block
    return pl.pallas_call(
        _expand_body,
        grid=(grid,),
        in_specs=[pl.BlockSpec((block, k), lambda i: (i, 0))],
        out_specs=pl.BlockSpec((block, k * k), lambda i: (i, 0)),
        out_shape=jax.ShapeDtypeStruct((n, k * k), jnp.float32),
    )(ls_flat)


def kernel(token_ids, mu_table, log_sigma_diag, phi_table):
    b, l = token_ids.shape
    k = mu_table.shape[1]
    p = phi_table.shape[1]
    n = b * l
    tok_flat = token_ids.reshape(n).astype(jnp.int32)
    phi_pad = jnp.pad(phi_table, ((0, 0), (0, 8 - p)))
    mu_flat, ls_flat, phi_flat = _sc_gather(
        tok_flat, mu_table, log_sigma_diag, phi_pad)
    sigma_flat = _expand_diag(ls_flat, block=512)
    mu = mu_flat.reshape(b, l, k)
    sigma = sigma_flat.reshape(b, l, k, k)
    phi = phi_flat[:, :p].reshape(b, l, p)
    return (mu, sigma, phi)
